# Initial kernel scaffold; baseline (speedup 1.0000x reference)
import jax
import jax.numpy as jnp
from jax.experimental import pallas as pl


def kernel(idx, table):
    raise NotImplementedError("write your pallas kernel here")


# speedup vs baseline: 1.0353x; 1.0353x over previous
import functools

import jax
import jax.numpy as jnp
from jax import lax
from jax.experimental import pallas as pl
from jax.experimental.pallas import tpu as pltpu
from jax.experimental.pallas import tpu_sc as plsc

VOCAB = 1000
B, T = 1024, 50
NROWS = B * T
NW = 32
BPW = NROWS // NW
C = 40
G = BPW // C


def _make_gather():
  mesh = plsc.VectorSubcoreMesh(core_axis_name="c", subcore_axis_name="s")

  @functools.partial(
      pl.kernel,
      out_type=jax.ShapeDtypeStruct((NROWS, VOCAB), jnp.float32),
      mesh=mesh,
      scratch_types=[
          pltpu.VMEM((G, C), jnp.int32),
          pltpu.VMEM((2, C, VOCAB), jnp.float32),
          pltpu.SemaphoreType.DMA,
          pltpu.SemaphoreType.DMA,
      ],
      compiler_params=pltpu.CompilerParams(use_tc_tiling_on_sc=False),
  )
  def k(idx_hbm, table_hbm, out_hbm, idx_v, rows_v, sem0, sem1):
    wid = lax.axis_index("s") * 2 + lax.axis_index("c")
    base = wid * BPW
    sems = (sem0, sem1)

    pltpu.sync_copy(idx_hbm.at[pl.ds(wid * G, G)], idx_v)

    def gather(g, b):
      return pltpu.make_async_copy(
          table_hbm.at[idx_v.at[g]], rows_v.at[b], sems[b])

    gather(0, 0).start()

    def outer(i2, _):
      for b in range(2):
        g = i2 * 2 + b
        nxt = g + 1

        @pl.when(nxt < G)
        def _():
          gather(nxt, 1 - b).start()

        gather(g, b).wait()
        pltpu.sync_copy(rows_v.at[b], out_hbm.at[pl.ds(base + g * C, C)])
      return _

    lax.fori_loop(0, G // 2, outer, None)

  return k


_gather = _make_gather()


def kernel(idx, table):
  idx2d = idx.reshape(NW * G, C).astype(jnp.int32)
  out = _gather(idx2d, table)
  return out.reshape(B, T, VOCAB)


# speedup vs baseline: 1.0370x; 1.0016x over previous
import functools

import jax
import jax.numpy as jnp
from jax import lax
from jax.experimental import pallas as pl
from jax.experimental.pallas import tpu as pltpu
from jax.experimental.pallas import tpu_sc as plsc

VOCAB = 1000
B, T = 1024, 50
NW = 32
BPW = B // NW


def _make_gather():
  mesh = plsc.VectorSubcoreMesh(core_axis_name="c", subcore_axis_name="s")

  @functools.partial(
      pl.kernel,
      out_type=jax.ShapeDtypeStruct((B, T, VOCAB), jnp.float32),
      mesh=mesh,
      scratch_types=[
          pltpu.VMEM((BPW, T), jnp.int32),
          pltpu.VMEM((2, T, VOCAB), jnp.float32),
          pltpu.SemaphoreType.DMA,
          pltpu.SemaphoreType.DMA,
      ],
      compiler_params=pltpu.CompilerParams(use_tc_tiling_on_sc=False),
  )
  def k(idx_hbm, table_hbm, out_hbm, idx_v, rows_v, sem0, sem1):
    wid = lax.axis_index("s") * 2 + lax.axis_index("c")
    base = wid * BPW
    sems = (sem0, sem1)

    pltpu.sync_copy(idx_hbm.at[pl.ds(base, BPW)], idx_v)

    def gather(c, b):
      return pltpu.make_async_copy(
          table_hbm.at[idx_v.at[c]], rows_v.at[b], sems[b])

    gather(0, 0).start()

    def outer(i2, _):
      for b in range(2):
        c = i2 * 2 + b
        nxt = c + 1

        @pl.when(nxt < BPW)
        def _():
          gather(nxt, 1 - b).start()

        gather(c, b).wait()
        pltpu.sync_copy(rows_v.at[b], out_hbm.at[base + c])
      return _

    lax.fori_loop(0, BPW // 2, outer, None)

  return k


_gather = _make_gather()


def kernel(idx, table):
  return _gather(idx.astype(jnp.int32), table)


# speedup vs baseline: 2.0742x; 2.0002x over previous
import functools

import jax
import jax.numpy as jnp
from jax import lax
from jax.experimental import pallas as pl
from jax.experimental.pallas import tpu as pltpu
from jax.experimental.pallas import tpu_sc as plsc

VOCAB = 1000
VPAD = 1024
B, T = 1024, 50
NW = 32
BPW = B // NW


def _make_gather():
  mesh = plsc.VectorSubcoreMesh(core_axis_name="c", subcore_axis_name="s")

  @functools.partial(
      pl.kernel,
      out_type=jax.ShapeDtypeStruct((B, T, VPAD), jnp.float32),
      mesh=mesh,
      scratch_types=[
          pltpu.VMEM((BPW, T), jnp.int32),
          pltpu.VMEM((2, T, VPAD), jnp.float32),
          pltpu.SemaphoreType.DMA,
          pltpu.SemaphoreType.DMA,
      ],
      compiler_params=pltpu.CompilerParams(use_tc_tiling_on_sc=True),
  )
  def k(idx_hbm, table_hbm, out_hbm, idx_v, rows_v, sem0, sem1):
    wid = lax.axis_index("s") * 2 + lax.axis_index("c")
    base = wid * BPW
    sems = (sem0, sem1)

    pltpu.sync_copy(idx_hbm.at[pl.ds(base, BPW)], idx_v)

    def gather(c, b):
      return pltpu.make_async_copy(
          table_hbm.at[idx_v.at[c]], rows_v.at[b], sems[b])

    gather(0, 0).start()

    def outer(i2, _):
      for b in range(2):
        c = i2 * 2 + b
        nxt = c + 1

        @pl.when(nxt < BPW)
        def _():
          gather(nxt, 1 - b).start()

        gather(c, b).wait()
        pltpu.sync_copy(rows_v.at[b], out_hbm.at[base + c])
      return _

    lax.fori_loop(0, BPW // 2, outer, None)

  return k


_gather = _make_gather()


def kernel(idx, table):
  table_p = jnp.pad(table, ((0, 0), (0, VPAD - VOCAB)))
  out_p = _gather(idx.astype(jnp.int32), table_p)
  return out_p[..., :VOCAB]
